# Initial kernel scaffold; baseline (speedup 1.0000x reference)
import jax
import jax.numpy as jnp
from jax.experimental import pallas as pl


def kernel(wild_x, wild_edge_index, wild_edge_attr, wild_batch, mutant_x, mutant_edge_index, mutant_edge_attr, mutant_batch, params):
    raise NotImplementedError("write your pallas kernel here")


# speedup vs baseline: 1.0233x; 1.0233x over previous
import functools
import jax
import jax.numpy as jnp
from jax.experimental import pallas as pl
from jax.experimental.pallas import tpu as pltpu

N = 10000
E = 160000
DIN = 256
H = 512
G = 64

ROWS = 1000


def _ln_rows(x, g, b):
    m = jnp.mean(x, axis=-1, keepdims=True)
    v = jnp.mean((x - m) ** 2, axis=-1, keepdims=True)
    return (x - m) * jax.lax.rsqrt(v + 1e-5) * g + b


def _in_proj_body(x_ref, g_ref, b_ref, w_ref, bp_ref, o_ref):
    x = _ln_rows(x_ref[...], g_ref[...], b_ref[...])
    o_ref[...] = jnp.dot(x, w_ref[...], preferred_element_type=jnp.float32) + bp_ref[...]


def _in_proj(x, g, b, w, bp):
    return pl.pallas_call(
        _in_proj_body,
        grid=(N // ROWS,),
        in_specs=[
            pl.BlockSpec((ROWS, DIN), lambda i: (i, 0)),
            pl.BlockSpec((DIN,), lambda i: (0,)),
            pl.BlockSpec((DIN,), lambda i: (0,)),
            pl.BlockSpec((DIN, H), lambda i: (0, 0)),
            pl.BlockSpec((H,), lambda i: (0,)),
        ],
        out_specs=pl.BlockSpec((ROWS, H), lambda i: (i, 0)),
        out_shape=jax.ShapeDtypeStruct((N, H), jnp.float32),
    )(x, g, b, w, bp)


def _act_body(y_ref, w1i_ref, w1o_ref, w2_ref, o_ref):
    y = y_ref[...]
    h_in = jnp.maximum(jnp.dot(y, w1i_ref[...], preferred_element_type=jnp.float32), 0.0)
    h_out = jnp.maximum(jnp.dot(y, w1o_ref[...], preferred_element_type=jnp.float32), 0.0)
    w2 = w2_ref[...]
    t_in = jnp.dot(h_in, w2[:, :2], preferred_element_type=jnp.float32)
    t_out = jnp.dot(h_out, w2[:, 2:], preferred_element_type=jnp.float32)
    o_ref[...] = jnp.concatenate([t_in, t_out], axis=-1)


def _act_t4(y, w1i, w1o, w2i, w2o):
    w2 = jnp.concatenate([w2i, w2o], axis=1)
    return pl.pallas_call(
        _act_body,
        grid=(N // ROWS,),
        in_specs=[
            pl.BlockSpec((ROWS, H), lambda i: (i, 0)),
            pl.BlockSpec((H, H), lambda i: (0, 0)),
            pl.BlockSpec((H, H), lambda i: (0, 0)),
            pl.BlockSpec((H, 4), lambda i: (0, 0)),
        ],
        out_specs=pl.BlockSpec((ROWS, 4), lambda i: (i, 0)),
        out_shape=jax.ShapeDtypeStruct((N, 4), jnp.float32),
    )(y, w1i, w1o, w2)


def _post_body(g512_ref, g16_ref, x_ref, ee_ref, we_ref, be_ref,
               pg_ref, pb_ref, fg_ref, fb_ref,
               w1_ref, b1_ref, w2_ref, b2_ref, og_ref, ob_ref, o_ref):
    agg = g512_ref[...] + jnp.dot(g16_ref[...], ee_ref[...], preferred_element_type=jnp.float32)
    out = jnp.maximum(jnp.dot(agg, we_ref[...], preferred_element_type=jnp.float32) + be_ref[...], 0.0)
    c = _ln_rows(out, pg_ref[...], pb_ref[...])
    c = _ln_rows(c, fg_ref[...], fb_ref[...])
    h = jnp.maximum(jnp.dot(c, w1_ref[...], preferred_element_type=jnp.float32) + b1_ref[...], 0.0)
    out = jnp.dot(h, w2_ref[...], preferred_element_type=jnp.float32) + b2_ref[...] + x_ref[...]
    o_ref[...] = _ln_rows(out, og_ref[...], ob_ref[...])


def _post(g512, g16, x, blk):
    ee = blk['edge_enc']
    we, be = blk['env']
    pg, pb = blk['post_ln']
    fg, fb = blk['feat_ln']
    w1, b1, w2, b2 = blk['enh']
    og, ob = blk['out_ln']
    vec = lambda: pl.BlockSpec((H,), lambda i: (0,))
    mat = lambda: pl.BlockSpec((H, H), lambda i: (0, 0))
    return pl.pallas_call(
        _post_body,
        grid=(N // ROWS,),
        in_specs=[
            pl.BlockSpec((ROWS, H), lambda i: (i, 0)),
            pl.BlockSpec((ROWS, 16), lambda i: (i, 0)),
            pl.BlockSpec((ROWS, H), lambda i: (i, 0)),
            pl.BlockSpec((16, H), lambda i: (0, 0)),
            mat(), vec(), vec(), vec(), vec(), vec(),
            mat(), vec(), mat(), vec(), vec(), vec(),
        ],
        out_specs=pl.BlockSpec((ROWS, H), lambda i: (i, 0)),
        out_shape=jax.ShapeDtypeStruct((N, H), jnp.float32),
    )(g512, g16, x, ee, we, be, pg, pb, fg, fb, w1, b1, w2, b2, og, ob)


def _head_body(w_ref, m_ref, w1_ref, b1_ref, w2_ref, b2_ref, w3_ref, b3_ref, o_ref):
    diff = m_ref[...] - w_ref[...]
    h = jnp.maximum(jnp.dot(diff, w1_ref[...], preferred_element_type=jnp.float32) + b1_ref[...], 0.0)
    h = jnp.maximum(jnp.dot(h, w2_ref[...], preferred_element_type=jnp.float32) + b2_ref[...], 0.0)
    o_ref[...] = jnp.dot(h, w3_ref[...], preferred_element_type=jnp.float32) + b3_ref[...]


def _head(wpool, mpool, head):
    w1, b1, w2, b2, w3, b3 = head
    o = pl.pallas_call(
        _head_body,
        out_shape=jax.ShapeDtypeStruct((G, 1), jnp.float32),
    )(wpool, mpool, w1, b1, w2, b2, w3, b3)
    return o[:, 0]


def _seg(vals, dst, n):
    return jax.ops.segment_sum(vals, dst, num_segments=n)


def _process(x, edge_index, ea, batch, params):
    src, dst = edge_index[0], edge_index[1]
    deg = _seg(jnp.ones((E,), jnp.float32), dst, N)
    rsd = jax.lax.rsqrt(jnp.clip(deg, 1.0, None))
    norm = rsd[src] * rsd[dst]
    g, b = params['in_ln']
    wp, bp = params['in_proj']
    x = _in_proj(x, g, b, wp, bp)
    for blk in params['blocks']:
        y = _seg(x[src] * norm[:, None], dst, N)
        w1i, w2i = blk['act_in']
        w1o, w2o = blk['act_out']
        t4 = _act_t4(y, w1i, w1o, w2i, w2o)
        l4 = _seg(t4[src] * norm[:, None], dst, N)
        p_in = jax.nn.sigmoid(l4[:, 0] - l4[:, 1])
        p_out = jax.nn.sigmoid(l4[:, 2] - l4[:, 3])
        ew = p_out[src] * p_in[dst] * norm
        g512 = _seg(x[src] * ew[:, None], dst, N)
        g16 = _seg(ea * ew[:, None], dst, N)
        x = _post(g512, g16, x, blk)
    return _seg(x, batch, G)


def kernel(wild_x, wild_edge_index, wild_edge_attr, wild_batch,
           mutant_x, mutant_edge_index, mutant_edge_attr, mutant_batch, params):
    w = _process(wild_x, wild_edge_index, wild_edge_attr, wild_batch, params)
    m = _process(mutant_x, mutant_edge_index, mutant_edge_attr, mutant_batch, params)
    return _head(w, m, params['head'])


# speedup vs baseline: 3.6930x; 3.6088x over previous
import functools
import jax
import jax.numpy as jnp
from jax import lax
from jax.experimental import pallas as pl
from jax.experimental.pallas import tpu as pltpu
from jax.experimental.pallas import tpu_sc as plsc

N = 10000
E = 160000
EPAD = 163840
DIN = 256
H = 512
G = 64
NP = 10240
TRASH = 10100
ROWS = 1000

_f32 = jnp.float32
_i32 = jnp.int32


def _mesh():
    return plsc.VectorSubcoreMesh(core_axis_name="c", subcore_axis_name="s",
                                  num_cores=2, num_subcores=16)


def _zero_buf(ref, rows, width):
    z = jnp.zeros((16,), _f32)
    for r in range(rows):
        for j in range(width // 16):
            ref[r, pl.ds(j * 16, 16)] = z


_B5 = 80
_EPT5 = EPAD // 16
_NB5 = _EPT5 // _B5
_K5 = 4
_NR5 = _NB5 // _K5


def _u512(tab4, src, dst, ept=_EPT5):
    nrounds = ept // (_K5 * _B5)

    @functools.partial(
        pl.kernel,
        out_type=jax.ShapeDtypeStruct((4, NP, 128), _f32),
        mesh=_mesh(),
        scratch_types=(
            pltpu.VMEM((_K5 * _B5,), _i32),
            pltpu.VMEM((_K5 * _B5,), _i32),
            [pltpu.VMEM((_B5,), _i32) for _ in range(_K5)],
            [pltpu.VMEM((_B5,), _i32) for _ in range(_K5)],
            [pltpu.VMEM((_B5, 128), _f32) for _ in range(_K5)],
            pltpu.VMEM((16, 128), _f32),
            pltpu.VMEM_SHARED((NP, 128), _f32),
            pltpu.SemaphoreType.DMA,
        ),
    )
    def k(tab_h, src_h, dst_h, out_h, sall, dall, idxb, dstb, rowsb, zbuf, acc, sem):
        c = lax.axis_index("c")
        s = lax.axis_index("s")
        base = s * ept
        _zero_buf(zbuf, 16, 128)
        stripe = NP // 16
        r0 = s * stripe
        rnd = _K5 * _B5
        for kk in range(2):
            kslice = c * 2 + kk
            for t in range(stripe // 16):
                pltpu.sync_copy(zbuf, acc.at[pl.ds(r0 + t * 16, 16)])
            plsc.subcore_barrier()

            def fill(q):
                off = q * _B5
                for j in range(_B5 // 16):
                    sl = pl.ds(j * 16, 16)
                    sv = sall[pl.ds(off + j * 16, 16)]
                    idxb[q][sl] = sv * 4 + kslice
                    dstb[q][sl] = dall[pl.ds(off + j * 16, 16)]

            def round_body(r, carry):
                e0 = base + r * rnd
                pltpu.sync_copy(src_h.at[pl.ds(e0, rnd)], sall)
                pltpu.sync_copy(dst_h.at[pl.ds(e0, rnd)], dall)
                handles = []
                for q in range(_K5):
                    fill(q)
                    handles.append(pltpu.async_copy(tab_h.at[idxb[q]], rowsb[q], sem))
                for h in handles:
                    h.wait()
                for q in range(_K5):
                    pltpu.sync_copy(rowsb[q], acc.at[dstb[q]], add=True)
                return carry

            lax.fori_loop(0, nrounds, round_body, 0)
            plsc.subcore_barrier()
            pltpu.sync_copy(acc.at[pl.ds(r0, stripe)], out_h.at[kslice, pl.ds(r0, stripe)])
            plsc.subcore_barrier()

    return k(tab4, src, dst)


_B6 = 80
_EPT6 = EPAD // 32
_NB6 = _EPT6 // _B6
_K6 = 4
_NR6 = _NB6 // _K6


def _u16(tab, src, dst):
    @functools.partial(
        pl.kernel,
        out_type=jax.ShapeDtypeStruct((2, NP, 16), _f32),
        mesh=_mesh(),
        scratch_types=(
            pltpu.VMEM((_EPT6,), _i32),
            pltpu.VMEM((_EPT6,), _i32),
            [pltpu.VMEM((_B6,), _i32) for _ in range(_K6)],
            [pltpu.VMEM((_B6,), _i32) for _ in range(_K6)],
            [pltpu.VMEM((_B6, 16), _f32) for _ in range(_K6)],
            pltpu.VMEM((_B6, 16), _f32),
            pltpu.VMEM_SHARED((NP, 16), _f32),
            pltpu.SemaphoreType.DMA,
        ),
        compiler_params=pltpu.CompilerParams(use_tc_tiling_on_sc=False),
    )
    def k(tab_h, src_h, dst_h, out_h, srcall, dstall, idxb, dstb, rowsb, zbuf, acc, sem):
        c = lax.axis_index("c")
        s = lax.axis_index("s")
        base = (c * 16 + s) * _EPT6
        pltpu.sync_copy(src_h.at[pl.ds(base, _EPT6)], srcall)
        pltpu.sync_copy(dst_h.at[pl.ds(base, _EPT6)], dstall)
        _zero_buf(zbuf, _B6, 16)
        stripe = NP // 16
        r0 = s * stripe
        for t in range(stripe // _B6):
            pltpu.sync_copy(zbuf, acc.at[pl.ds(r0 + t * _B6, _B6)])
        plsc.subcore_barrier()

        def fill(bi, q):
            off = bi * _B6
            for j in range(_B6 // 16):
                sl = pl.ds(j * 16, 16)
                idxb[q][sl] = srcall[pl.ds(off + j * 16, 16)]
                dstb[q][sl] = dstall[pl.ds(off + j * 16, 16)]

        def round_body(r, carry):
            handles = []
            for q in range(_K6):
                fill(r * _K6 + q, q)
                handles.append(pltpu.async_copy(tab_h.at[idxb[q]], rowsb[q], sem))
            for h in handles:
                h.wait()
            for q in range(_K6):
                pltpu.sync_copy(rowsb[q], acc.at[dstb[q]], add=True)
            return carry

        lax.fori_loop(0, _NR6, round_body, 0)
        plsc.subcore_barrier()
        pltpu.sync_copy(acc.at[pl.ds(r0, stripe)], out_h.at[c, pl.ds(r0, stripe)])

    return k(tab, src, dst)


_ECH = 2560


def _escale(eaflat, sout, src):
    @functools.partial(
        pl.kernel,
        out_type=jax.ShapeDtypeStruct((EPAD * 16,), _f32),
        mesh=_mesh(),
        scratch_types=(
            pltpu.VMEM((_ECH * 16,), _f32),
            pltpu.VMEM((_ECH,), _i32),
            pltpu.VMEM((N,), _f32),
        ),
        compiler_params=pltpu.CompilerParams(use_tc_tiling_on_sc=False,
                                             needs_layout_passes=False),
    )
    def k(ea_h, sout_h, src_h, out_h, eav, srcall, soutv):
        c = lax.axis_index("c")
        s = lax.axis_index("s")
        base_e = (c * 16 + s) * _EPT6
        pltpu.sync_copy(sout_h, soutv)
        for ch in range(_EPT6 // _ECH):
            e0 = base_e + ch * _ECH
            pltpu.sync_copy(src_h.at[pl.ds(e0, _ECH)], srcall)
            pltpu.sync_copy(ea_h.at[pl.ds(e0 * 16, _ECH * 16)], eav)

            def grp(g, carry):
                s16 = srcall[pl.ds(g * 16, 16)]
                w16 = plsc.load_gather(soutv, [s16])
                for rr in range(16):
                    off = g * 256 + rr * 16
                    eav[pl.ds(off, 16)] = eav[pl.ds(off, 16)] * w16[rr]
                return carry

            lax.fori_loop(0, _ECH // 16, grp, 0)
            pltpu.sync_copy(eav, out_h.at[pl.ds(e0 * 16, _ECH * 16)])

    return k(eaflat, sout, src)


def _ln_rows(x, g, b):
    m = jnp.mean(x, axis=-1, keepdims=True)
    v = jnp.mean((x - m) ** 2, axis=-1, keepdims=True)
    return (x - m) * lax.rsqrt(v + 1e-5) * g + b


def _vec(n=H):
    return pl.BlockSpec((n,), lambda i: (0,))


def _mat(a=H, b=H):
    return pl.BlockSpec((a, b), lambda i: (0, 0))


def _rblk(w):
    return pl.BlockSpec((ROWS, w), lambda i: (i, 0))


def _ublk(q):
    return pl.BlockSpec((1, ROWS, 128), lambda i, q=q: (q, i, 0))


def _hblk(q, w=16):
    return pl.BlockSpec((1, ROWS, w), lambda i, q=q: (q, i, 0))


def _in_proj_body(x_ref, d0_ref, d1_ref, g_ref, b_ref, w_ref, bp_ref,
                  x0_ref, xs_ref, rsd_ref):
    deg = d0_ref[0][:, 0] + d1_ref[0][:, 0]
    rsd = lax.rsqrt(jnp.maximum(deg, 1.0))
    x = _ln_rows(x_ref[...], g_ref[...], b_ref[...])
    x0 = jnp.dot(x, w_ref[...], preferred_element_type=_f32) + bp_ref[...]
    x0_ref[...] = x0
    xs_ref[...] = x0 * rsd[:, None]
    rsd_ref[...] = rsd[:, None]


def _in_proj(x, degh, g, b, w, bp):
    return pl.pallas_call(
        _in_proj_body,
        grid=(N // ROWS,),
        in_specs=[_rblk(DIN), _hblk(0), _hblk(1), _vec(DIN), _vec(DIN),
                  _mat(DIN, H), _vec(H)],
        out_specs=(_rblk(H), _rblk(H), pl.BlockSpec((ROWS, 1), lambda i: (i, 0))),
        out_shape=(jax.ShapeDtypeStruct((N, H), _f32),
                   jax.ShapeDtypeStruct((N, H), _f32),
                   jax.ShapeDtypeStruct((N, 1), _f32)),
    )(x, degh, degh, g, b, w, bp)


def _act_body(u0, u1, u2, u3, rsd_ref, w1i_ref, w1o_ref, w2_ref, o_ref):
    rsd = rsd_ref[...]
    us = (u0[0], u1[0], u2[0], u3[0])
    w1i = w1i_ref[...]
    w1o = w1o_ref[...]
    hi = jnp.zeros((ROWS, H), _f32)
    ho = jnp.zeros((ROWS, H), _f32)
    for q in range(4):
        y = us[q] * rsd
        hi = hi + jnp.dot(y, w1i[q * 128:(q + 1) * 128, :], preferred_element_type=_f32)
        ho = ho + jnp.dot(y, w1o[q * 128:(q + 1) * 128, :], preferred_element_type=_f32)
    hi = jnp.maximum(hi, 0.0)
    ho = jnp.maximum(ho, 0.0)
    w2 = w2_ref[...]
    t = jnp.concatenate([
        jnp.dot(hi, w2[:, :2], preferred_element_type=_f32),
        jnp.dot(ho, w2[:, 2:], preferred_element_type=_f32),
    ], axis=-1) * rsd
    o_ref[...] = jnp.concatenate([t, jnp.zeros((ROWS, 12), _f32)], axis=-1)


def _act(u, rsd, w1i, w1o, w2i, w2o):
    w2 = jnp.concatenate([w2i, w2o], axis=1)
    return pl.pallas_call(
        _act_body,
        grid=(N // ROWS,),
        in_specs=[_ublk(0), _ublk(1), _ublk(2), _ublk(3),
                  pl.BlockSpec((ROWS, 1), lambda i: (i, 0)),
                  _mat(), _mat(), _mat(H, 4)],
        out_specs=_rblk(16),
        out_shape=jax.ShapeDtypeStruct((N, 16), _f32),
    )(u, u, u, u, rsd, w1i, w1o, w2)


def _gate_body(l0_ref, l1_ref, rsd_ref, x_ref, si_ref, so_ref, xs_ref):
    la = l0_ref[0] + l1_ref[0]
    rsd = rsd_ref[...][:, 0]
    p_in = jax.nn.sigmoid(rsd * (la[:, 0] - la[:, 1]))
    p_out = jax.nn.sigmoid(rsd * (la[:, 2] - la[:, 3]))
    s_in = p_in * rsd
    s_out = p_out * rsd
    si_ref[...] = s_in[:, None]
    so_ref[...] = s_out[:, None]
    xs_ref[...] = x_ref[...] * s_out[:, None]


def _gate(l, rsd, x):
    return pl.pallas_call(
        _gate_body,
        grid=(N // ROWS,),
        in_specs=[_hblk(0), _hblk(1),
                  pl.BlockSpec((ROWS, 1), lambda i: (i, 0)), _rblk(H)],
        out_specs=(pl.BlockSpec((ROWS, 1), lambda i: (i, 0)),
                   pl.BlockSpec((ROWS, 1), lambda i: (i, 0)),
                   _rblk(H)),
        out_shape=(jax.ShapeDtypeStruct((N, 1), _f32),
                   jax.ShapeDtypeStruct((N, 1), _f32),
                   jax.ShapeDtypeStruct((N, H), _f32)),
    )(l, l, rsd, x)


def _post_body(u0, u1, u2, u3, e0_ref, e1_ref, si_ref, rsd_ref, x_ref,
               ee_ref, we_ref, be_ref, pg_ref, pb_ref, fg_ref, fb_ref,
               w1_ref, b1_ref, w2_ref, b2_ref, og_ref, ob_ref,
               nx_ref, nxs_ref):
    si = si_ref[...]
    u = jnp.concatenate([u0[0], u1[0], u2[0], u3[0]], axis=-1) * si
    e = (e0_ref[0] + e1_ref[0]) * si
    agg = u + jnp.dot(e, ee_ref[...], preferred_element_type=_f32)
    out = jnp.maximum(jnp.dot(agg, we_ref[...], preferred_element_type=_f32) + be_ref[...], 0.0)
    cc = _ln_rows(out, pg_ref[...], pb_ref[...])
    cc = _ln_rows(cc, fg_ref[...], fb_ref[...])
    h = jnp.maximum(jnp.dot(cc, w1_ref[...], preferred_element_type=_f32) + b1_ref[...], 0.0)
    out = jnp.dot(h, w2_ref[...], preferred_element_type=_f32) + b2_ref[...] + x_ref[...]
    nx = _ln_rows(out, og_ref[...], ob_ref[...])
    nx_ref[...] = nx
    nxs_ref[...] = nx * rsd_ref[...]


def _post(u, e16, si, rsd, x, blk):
    we, be = blk['env']
    pg, pb = blk['post_ln']
    fg, fb = blk['feat_ln']
    w1, b1, w2, b2 = blk['enh']
    og, ob = blk['out_ln']
    return pl.pallas_call(
        _post_body,
        grid=(N // ROWS,),
        in_specs=[_ublk(0), _ublk(1), _ublk(2), _ublk(3), _hblk(0), _hblk(1),
                  pl.BlockSpec((ROWS, 1), lambda i: (i, 0)),
                  pl.BlockSpec((ROWS, 1), lambda i: (i, 0)), _rblk(H),
                  _mat(16, H), _mat(), _vec(), _vec(), _vec(), _vec(), _vec(),
                  _mat(), _vec(), _mat(), _vec(), _vec(), _vec()],
        out_specs=(_rblk(H), _rblk(H)),
        out_shape=(jax.ShapeDtypeStruct((N, H), _f32),
                   jax.ShapeDtypeStruct((N, H), _f32)),
    )(u, u, u, u, e16, e16, si, rsd, x, blk['edge_enc'], we, be,
      pg, pb, fg, fb, w1, b1, w2, b2, og, ob)


def _head_body(w0, w1r, w2r, w3r, m0, m1r, m2r, m3r,
               w1_ref, b1_ref, w2_ref, b2_ref, w3_ref, b3_ref, o_ref):
    wp = jnp.concatenate([w0[0], w1r[0], w2r[0], w3r[0]], axis=-1)
    mp = jnp.concatenate([m0[0], m1r[0], m2r[0], m3r[0]], axis=-1)
    diff = mp - wp
    h = jnp.maximum(jnp.dot(diff, w1_ref[...], preferred_element_type=_f32) + b1_ref[...], 0.0)
    h = jnp.maximum(jnp.dot(h, w2_ref[...], preferred_element_type=_f32) + b2_ref[...], 0.0)
    o_ref[...] = jnp.dot(h, w3_ref[...], preferred_element_type=_f32) + b3_ref[...]


def _gblk(q):
    return pl.BlockSpec((1, G, 128), lambda i, q=q: (q, 0, 0))


def _head(wpool, mpool, head):
    w1, b1, w2, b2, w3, b3 = head
    o = pl.pallas_call(
        _head_body,
        grid=(1,),
        in_specs=[_gblk(0), _gblk(1), _gblk(2), _gblk(3),
                  _gblk(0), _gblk(1), _gblk(2), _gblk(3),
                  pl.BlockSpec((H, 2 * H), lambda i: (0, 0)),
                  pl.BlockSpec((2 * H,), lambda i: (0,)),
                  pl.BlockSpec((2 * H, H), lambda i: (0, 0)),
                  pl.BlockSpec((H,), lambda i: (0,)),
                  pl.BlockSpec((H, 1), lambda i: (0, 0)),
                  pl.BlockSpec((1,), lambda i: (0,))],
        out_specs=pl.BlockSpec((G, 1), lambda i: (0, 0)),
        out_shape=jax.ShapeDtypeStruct((G, 1), _f32),
    )(wpool, wpool, wpool, wpool, mpool, mpool, mpool, mpool,
      w1, b1, w2, b2, w3, b3)
    return o[:, 0]


def _graph(x, edge_index, ea, batch, params):
    src = edge_index[0]
    dst = edge_index[1]
    padn = EPAD - E
    srcp = jnp.concatenate([src, jnp.zeros((padn,), _i32)])
    dstp = jnp.concatenate([dst, jnp.full((padn,), TRASH, _i32)])
    eaflat = jnp.concatenate([ea, jnp.zeros((padn, 16), _f32)]).reshape(EPAD * 16)
    eids = jnp.arange(EPAD, dtype=_i32)
    ones16 = jnp.ones((N, 16), _f32)
    degh = _u16(ones16, srcp, dstp)
    g, b = params['in_ln']
    wp, bp = params['in_proj']
    x0, xs, rsd = _in_proj(x, degh, g, b, wp, bp)
    xcur, xscur = x0, xs
    for blk in params['blocks']:
        u1 = _u512(xscur.reshape(N * 4, 128), srcp, dstp)
        w1i, w2i = blk['act_in']
        w1o, w2o = blk['act_out']
        t4 = _act(u1, rsd, w1i, w1o, w2i, w2o)
        l = _u16(t4, srcp, dstp)
        s_in, s_out, xs3 = _gate(l, rsd, xcur)
        u3 = _u512(xs3.reshape(N * 4, 128), srcp, dstp)
        eascaled = _escale(eaflat, s_out.reshape(N), srcp)
        e16 = _u16(eascaled.reshape(EPAD, 16), eids, dstp)
        xcur, xscur = _post(u3, e16, s_in, rsd, xcur, blk)
    srcpool = jnp.minimum(jnp.arange(NP, dtype=_i32), N - 1)
    dstpool = jnp.concatenate([batch, jnp.full((NP - N,), TRASH, _i32)])
    return _u512(xcur.reshape(N * 4, 128), srcpool, dstpool, ept=NP // 16)


def kernel(wild_x, wild_edge_index, wild_edge_attr, wild_batch,
           mutant_x, mutant_edge_index, mutant_edge_attr, mutant_batch, params):
    w = _graph(wild_x, wild_edge_index, wild_edge_attr, wild_batch, params)
    m = _graph(mutant_x, mutant_edge_index, mutant_edge_attr, mutant_batch, params)
    return _head(w, m, params['head'])
